# baseline (speedup 1.0000x reference)
import functools

import jax
import jax.numpy as jnp
from jax import lax
from jax.experimental import pallas as pl
from jax.experimental.pallas import tpu as pltpu
from jax.experimental.pallas import tpu_sc as plsc

_NW = 32
_C = 8
_NBUF = 4


def _gather_combine(nodes2d, neigh2d, features, B, S, D, nch):
    n_real = B // _C
    cs = _C * S
    mesh = plsc.VectorSubcoreMesh(
        core_axis_name="c", subcore_axis_name="s", num_cores=2, num_subcores=16
    )

    @functools.partial(
        pl.kernel,
        out_type=jax.ShapeDtypeStruct((B, 2 * D), jnp.float32),
        mesh=mesh,
        scratch_types=[
            pltpu.VMEM((nch, _C), jnp.int32),
            pltpu.VMEM((nch, cs), jnp.int32),
        ]
        + [pltpu.VMEM((cs, D), jnp.float32)] * _NBUF
        + [pltpu.VMEM((_C, D), jnp.float32)] * _NBUF
        + [pltpu.VMEM((_C, 2 * D), jnp.float32)] * _NBUF
        + [pltpu.SemaphoreType.DMA] * (3 * _NBUF),
    )
    def k(nodes_hbm, neigh_hbm, feat_hbm, out_hbm, idxs_v, idxn_v, *bufs):
        rn = bufs[0:_NBUF]
        rs = bufs[_NBUF:2 * _NBUF]
        ov = bufs[2 * _NBUF:3 * _NBUF]
        semn = bufs[3 * _NBUF:4 * _NBUF]
        sems = bufs[4 * _NBUF:5 * _NBUF]
        semo = bufs[5 * _NBUF:6 * _NBUF]

        wid = lax.axis_index("s") * 2 + lax.axis_index("c")
        n_my = jnp.minimum(nch, jnp.maximum(0, n_real - nch * wid))
        row0 = wid * (nch * _C)

        pltpu.sync_copy(nodes_hbm.at[pl.ds(wid * nch, nch)], idxs_v)
        pltpu.sync_copy(neigh_hbm.at[pl.ds(wid * nch, nch)], idxn_v)

        def fire(c, b):
            cc = jnp.minimum(c, nch - 1)

            @pl.when(c < n_my)
            def _():
                pltpu.async_copy(
                    feat_hbm.at[idxn_v.at[cc, pl.ds(0, 64)]],
                    rn[b].at[pl.ds(0, 64)], semn[b])
                pltpu.async_copy(
                    feat_hbm.at[idxn_v.at[cc, pl.ds(64, 64)]],
                    rn[b].at[pl.ds(64, 64)], semn[b])
                pltpu.async_copy(feat_hbm.at[idxs_v.at[cc]], rs[b], sems[b])

        def wait_gather(c, b):
            cc = jnp.minimum(c, nch - 1)

            @pl.when(c < n_my)
            def _():
                pltpu.make_async_copy(
                    feat_hbm.at[idxn_v.at[cc, pl.ds(0, 64)]],
                    rn[b].at[pl.ds(0, 64)], semn[b]).wait()
                pltpu.make_async_copy(
                    feat_hbm.at[idxn_v.at[cc, pl.ds(64, 64)]],
                    rn[b].at[pl.ds(64, 64)], semn[b]).wait()
                pltpu.make_async_copy(feat_hbm.at[idxs_v.at[cc]], rs[b], sems[b]).wait()

        def wait_store(c, b):
            @pl.when((c >= 0) & (c < n_my))
            def _():
                pltpu.make_async_copy(ov[b], out_hbm.at[pl.ds(0, _C)], semo[b]).wait()

        def compute_store(c, b):
            @pl.when(c < n_my)
            def _():
                @plsc.parallel_loop(0, _C, unroll=2)
                def row(r):
                    for dd in range(D // 16):
                        sl = pl.ds(dd * 16, 16)
                        ov[b][r, sl] = rs[b][r, sl]
                        t = [
                            rn[b][r * S + 2 * s, sl] + rn[b][r * S + 2 * s + 1, sl]
                            for s in range(S // 2)
                        ]
                        while len(t) > 1:
                            t = [t[2 * i] + t[2 * i + 1] for i in range(len(t) // 2)]
                        ov[b][r, pl.ds(D + dd * 16, 16)] = t[0]
                pltpu.async_copy(ov[b], out_hbm.at[pl.ds(row0 + c * _C, _C)], semo[b])

        for b in range(_NBUF):
            fire(jnp.int32(b), b)

        def body(i, carry):
            jo = i * _NBUF
            for b in range(_NBUF):
                c = jo + b
                wait_gather(c, b)
                wait_store(c - _NBUF, b)
                compute_store(c, b)
                fire(c + _NBUF, b)
            return carry

        lax.fori_loop(0, nch // _NBUF, body, 0)
        for b in range(_NBUF):
            wait_store(jnp.int32(nch - _NBUF + b), b)

    return k(nodes2d, neigh2d, features)


def _matmul_relu(combined, weight, B, D, E):
    R = 4000 if B % 4000 == 0 else 2000

    def mm(x_ref, w_ref, o_ref):
        acc = lax.dot_general(
            x_ref[...], w_ref[...], (((1,), (1,)), ((), ())),
            preferred_element_type=jnp.float32,
        )
        o_ref[...] = jnp.maximum(acc, 0.0)

    return pl.pallas_call(
        mm,
        grid=(B // R,),
        in_specs=[
            pl.BlockSpec((R, 2 * D), lambda i: (i, 0)),
            pl.BlockSpec((E, 2 * D), lambda i: (0, 0)),
        ],
        out_specs=pl.BlockSpec((R, E), lambda i: (i, 0)),
        out_shape=jax.ShapeDtypeStruct((B, E), jnp.float32),
    )(combined, weight)


def _sc_half(nodes_h, neigh_h, features, B, S, D):
    nch = -(-B // (_NW * _C))
    nch = -(-nch // 8) * 8
    b_pad = _NW * nch * _C
    nodes_p = jnp.pad(nodes_h.astype(jnp.int32), (0, b_pad - B)).reshape(-1, _C)
    neigh_p = jnp.pad(
        neigh_h.astype(jnp.int32).reshape(-1), (0, (b_pad - B) * S)
    ).reshape(-1, _C * S)
    return _gather_combine(nodes_p, neigh_p, features, B, S, D, nch)


def kernel(nodes, neigh_idx, features, weight):
    B, S = neigh_idx.shape
    D = features.shape[1]
    E = weight.shape[0]
    scale = jnp.concatenate(
        [jnp.ones((D,), jnp.float32), jnp.full((D,), 1.0 / S, jnp.float32)]
    )
    w_used = weight * scale[None, :]
    combined = _sc_half(nodes, neigh_idx, features, B, S, D)
    return _matmul_relu(combined, w_used, B, D, E)


# speedup vs baseline: 1.0017x; 1.0017x over previous
import functools

import jax
import jax.numpy as jnp
from jax import lax
from jax.experimental import pallas as pl
from jax.experimental.pallas import tpu as pltpu
from jax.experimental.pallas import tpu_sc as plsc

_NW = 32
_C = 8
_NBUF = 4


def _gather_combine(nodes2d, neigh2d, features, B, S, D, nch):
    n_real = B // _C
    cs = _C * S
    mesh = plsc.VectorSubcoreMesh(
        core_axis_name="c", subcore_axis_name="s", num_cores=2, num_subcores=16
    )

    @functools.partial(
        pl.kernel,
        out_type=jax.ShapeDtypeStruct((B, 2 * D), jnp.float32),
        mesh=mesh,
        scratch_types=[
            pltpu.VMEM((nch, _C), jnp.int32),
            pltpu.VMEM((nch, cs), jnp.int32),
        ]
        + [pltpu.VMEM((cs, D), jnp.float32)] * _NBUF
        + [pltpu.VMEM((_C, D), jnp.float32)] * _NBUF
        + [pltpu.VMEM((_C, 2 * D), jnp.float32)] * _NBUF
        + [pltpu.SemaphoreType.DMA] * (3 * _NBUF),
    )
    def k(nodes_hbm, neigh_hbm, feat_hbm, out_hbm, idxs_v, idxn_v, *bufs):
        rn = bufs[0:_NBUF]
        rs = bufs[_NBUF:2 * _NBUF]
        ov = bufs[2 * _NBUF:3 * _NBUF]
        semn = bufs[3 * _NBUF:4 * _NBUF]
        sems = bufs[4 * _NBUF:5 * _NBUF]
        semo = bufs[5 * _NBUF:6 * _NBUF]

        wid = lax.axis_index("s") * 2 + lax.axis_index("c")
        n_my = jnp.minimum(nch, jnp.maximum(0, n_real - nch * wid))
        row0 = wid * (nch * _C)

        pltpu.sync_copy(nodes_hbm.at[pl.ds(wid * nch, nch)], idxs_v)
        pltpu.sync_copy(neigh_hbm.at[pl.ds(wid * nch, nch)], idxn_v)

        def fire(c, b):
            cc = jnp.minimum(c, nch - 1)

            @pl.when(c < n_my)
            def _():
                pltpu.async_copy(feat_hbm.at[idxn_v.at[cc]], rn[b], semn[b])
                pltpu.async_copy(feat_hbm.at[idxs_v.at[cc]], rs[b], sems[b])

        def wait_gather(c, b):
            cc = jnp.minimum(c, nch - 1)

            @pl.when(c < n_my)
            def _():
                pltpu.make_async_copy(feat_hbm.at[idxn_v.at[cc]], rn[b], semn[b]).wait()
                pltpu.make_async_copy(feat_hbm.at[idxs_v.at[cc]], rs[b], sems[b]).wait()

        def wait_store(c, b):
            @pl.when((c >= 0) & (c < n_my))
            def _():
                pltpu.make_async_copy(ov[b], out_hbm.at[pl.ds(0, _C)], semo[b]).wait()

        def compute_store(c, b):
            @pl.when(c < n_my)
            def _():
                @plsc.parallel_loop(0, _C, unroll=2)
                def row(r):
                    for dd in range(D // 16):
                        sl = pl.ds(dd * 16, 16)
                        ov[b][r, sl] = rs[b][r, sl]
                        t = [
                            rn[b][r * S + 2 * s, sl] + rn[b][r * S + 2 * s + 1, sl]
                            for s in range(S // 2)
                        ]
                        while len(t) > 1:
                            t = [t[2 * i] + t[2 * i + 1] for i in range(len(t) // 2)]
                        ov[b][r, pl.ds(D + dd * 16, 16)] = t[0]
                pltpu.async_copy(ov[b], out_hbm.at[pl.ds(row0 + c * _C, _C)], semo[b])

        for b in range(_NBUF):
            fire(jnp.int32(b), b)

        def body(i, carry):
            jo = i * _NBUF
            for b in range(_NBUF):
                c = jo + b
                wait_gather(c, b)
                wait_store(c - _NBUF, b)
                compute_store(c, b)
                fire(c + _NBUF, b)
            return carry

        lax.fori_loop(0, nch // _NBUF, body, 0)
        for b in range(_NBUF):
            wait_store(jnp.int32(nch - _NBUF + b), b)

    return k(nodes2d, neigh2d, features)


def _matmul_relu(combined, weight, B, D, E):
    R = 4000 if B % 4000 == 0 else 2000

    def mm(x_ref, w_ref, o_ref):
        acc = lax.dot_general(
            x_ref[...], w_ref[...], (((1,), (1,)), ((), ())),
            preferred_element_type=jnp.float32,
        )
        o_ref[...] = jnp.maximum(acc, 0.0)

    return pl.pallas_call(
        mm,
        grid=(B // R,),
        in_specs=[
            pl.BlockSpec((R, 2 * D), lambda i: (i, 0)),
            pl.BlockSpec((E, 2 * D), lambda i: (0, 0)),
        ],
        out_specs=pl.BlockSpec((R, E), lambda i: (i, 0)),
        out_shape=jax.ShapeDtypeStruct((B, E), jnp.float32),
    )(combined, weight)


def _sc_half(nodes_h, neigh_h, features, B, S, D):
    nch = -(-B // (_NW * _C))
    nch = -(-nch // 8) * 8
    b_pad = _NW * nch * _C
    nodes_p = jnp.pad(nodes_h.astype(jnp.int32), (0, b_pad - B)).reshape(-1, _C)
    neigh_p = jnp.pad(
        neigh_h.astype(jnp.int32).reshape(-1), (0, (b_pad - B) * S)
    ).reshape(-1, _C * S)
    return _gather_combine(nodes_p, neigh_p, features, B, S, D, nch)


def kernel(nodes, neigh_idx, features, weight):
    B, S = neigh_idx.shape
    D = features.shape[1]
    E = weight.shape[0]
    scale = jnp.concatenate(
        [jnp.ones((D,), jnp.float32), jnp.full((D,), 1.0 / S, jnp.float32)]
    )
    w_used = weight * scale[None, :]
    combined = _sc_half(nodes, neigh_idx, features, B, S, D)
    return _matmul_relu(combined, w_used, B, D, E)
